# initial kernel scaffold (unmeasured)
import jax
import jax.numpy as jnp
from jax import lax
from jax.experimental import pallas as pl
from jax.experimental.pallas import tpu as pltpu


def kernel(
    x,
):
    def body(*refs):
        pass

    out_shape = jax.ShapeDtypeStruct(..., jnp.float32)
    return pl.pallas_call(body, out_shape=out_shape)(...)


# baseline (device time: 434471 ns/iter reference)
import jax
import jax.numpy as jnp
from jax import lax
from jax.experimental import pallas as pl
from jax.experimental.pallas import tpu as pltpu

N_CHUNKS = 8


def kernel(x):
    m, n = x.shape
    half = m // 2
    chunk = half // N_CHUNKS

    def body(x_hbm, out_hbm, in_vmem, send_x, recv_x, sum_buf,
             load_sem, store_sem, sx_send, sx_recv, sy_send, sy_recv):
        my_x = lax.axis_index("x")
        my_y = lax.axis_index("y")
        row0 = my_y * half

        for c in range(N_CHUNKS):
            slot = c % 2
            r = row0 + c * chunk

            cp = pltpu.make_async_copy(
                x_hbm.at[pl.ds(r, chunk), :], in_vmem, load_sem)
            cp.start()
            cp.wait()
            send_x[...] = in_vmem[...].astype(jnp.bfloat16)

            rdma_x = pltpu.make_async_remote_copy(
                src_ref=send_x,
                dst_ref=recv_x.at[slot],
                send_sem=sx_send.at[slot],
                recv_sem=sx_recv.at[slot],
                device_id=(1 - my_x, my_y),
                device_id_type=pl.DeviceIdType.MESH,
            )
            rdma_x.start()
            rdma_x.wait()

            sum_buf[...] = send_x[...] + recv_x[slot]

            st = pltpu.make_async_copy(
                sum_buf, out_hbm.at[pl.ds(r, chunk), :], store_sem)
            st.start()

            rdma_y = pltpu.make_async_remote_copy(
                src_ref=sum_buf,
                dst_ref=out_hbm.at[pl.ds(r, chunk), :],
                send_sem=sy_send.at[slot],
                recv_sem=sy_recv.at[slot],
                device_id=(my_x, 1 - my_y),
                device_id_type=pl.DeviceIdType.MESH,
            )
            rdma_y.start()
            rdma_y.wait()
            st.wait()

    return pl.pallas_call(
        body,
        out_shape=jax.ShapeDtypeStruct((m, n), jnp.bfloat16),
        in_specs=[pl.BlockSpec(memory_space=pltpu.MemorySpace.HBM)],
        out_specs=pl.BlockSpec(memory_space=pltpu.MemorySpace.HBM),
        scratch_shapes=[
            pltpu.VMEM((chunk, n), jnp.float32),
            pltpu.VMEM((chunk, n), jnp.bfloat16),
            pltpu.VMEM((2, chunk, n), jnp.bfloat16),
            pltpu.VMEM((chunk, n), jnp.bfloat16),
            pltpu.SemaphoreType.DMA,
            pltpu.SemaphoreType.DMA,
            pltpu.SemaphoreType.DMA((2,)),
            pltpu.SemaphoreType.DMA((2,)),
            pltpu.SemaphoreType.DMA((2,)),
            pltpu.SemaphoreType.DMA((2,)),
        ],
    )(x)


# device time: 235863 ns/iter; 1.8420x vs baseline; 1.8420x over previous
import jax
import jax.numpy as jnp
from jax import lax
from jax.experimental import pallas as pl
from jax.experimental.pallas import tpu as pltpu

N_CHUNKS = 8


def kernel(x):
    m, n = x.shape
    half = m // 2
    chunk = half // N_CHUNKS
    N = N_CHUNKS

    def body(x_hbm, out_hbm, in_vmem, send_x, recv_x, sum_buf,
             load_sems, store_sems, sx_send, sx_recv, sy_send, sy_recv,
             credit_x, credit_y):
        my_x = lax.axis_index("x")
        my_y = lax.axis_index("y")
        row0 = my_y * half
        x_peer = (1 - my_x, my_y)
        y_peer = (my_x, 1 - my_y)

        def load(c):
            return pltpu.make_async_copy(
                x_hbm.at[pl.ds(row0 + c * chunk, chunk), :],
                in_vmem.at[c % 2], load_sems.at[c % 2])

        def cast(c):
            send_x[c % 2] = in_vmem[c % 2].astype(jnp.bfloat16)

        def rdma_x(c):
            return pltpu.make_async_remote_copy(
                src_ref=send_x.at[c % 2], dst_ref=recv_x.at[c % 2],
                send_sem=sx_send.at[c % 2], recv_sem=sx_recv.at[c % 2],
                device_id=x_peer, device_id_type=pl.DeviceIdType.MESH)

        def rdma_y(c):
            return pltpu.make_async_remote_copy(
                src_ref=sum_buf.at[c % 2],
                dst_ref=out_hbm.at[pl.ds(row0 + c * chunk, chunk), :],
                send_sem=sy_send.at[c % 2], recv_sem=sy_recv.at[c % 2],
                device_id=y_peer, device_id_type=pl.DeviceIdType.MESH)

        def store(c):
            return pltpu.make_async_copy(
                sum_buf.at[c % 2],
                out_hbm.at[pl.ds(row0 + c * chunk, chunk), :],
                store_sems.at[c % 2])

        load(0).start()
        load(0).wait()
        cast(0)
        load(1).start()
        rdma_x(0).start()
        load(1).wait()
        cast(1)
        load(2).start()

        for c in range(N):
            if c + 1 < N:
                if c + 1 >= 2:
                    pl.semaphore_wait(credit_x, 1)
                rdma_x(c + 1).start()

            rdma_x(c).wait_recv()

            if c >= 2:
                rdma_y(c - 2).wait_send()
                store(c - 2).wait()

            sum_buf[c % 2] = send_x[c % 2] + recv_x[c % 2]
            if c <= N - 3:
                pl.semaphore_signal(
                    credit_x, inc=1, device_id=x_peer,
                    device_id_type=pl.DeviceIdType.MESH)

            if c >= 2:
                pl.semaphore_wait(credit_y, 1)
            store(c).start()
            rdma_y(c).start()

            if c >= 1:
                rdma_y(c - 1).wait_recv()
                if c - 1 <= N - 3:
                    pl.semaphore_signal(
                        credit_y, inc=1, device_id=y_peer,
                        device_id_type=pl.DeviceIdType.MESH)

            if c + 2 < N:
                load(c + 2).wait()
                rdma_x(c).wait_send()
                cast(c + 2)
                if c + 3 < N:
                    load(c + 3).start()

        rdma_y(N - 1).wait_recv()
        rdma_x(N - 2).wait_send()
        rdma_x(N - 1).wait_send()
        rdma_y(N - 2).wait_send()
        rdma_y(N - 1).wait_send()
        store(N - 2).wait()
        store(N - 1).wait()

    return pl.pallas_call(
        body,
        out_shape=jax.ShapeDtypeStruct((m, n), jnp.bfloat16),
        in_specs=[pl.BlockSpec(memory_space=pltpu.MemorySpace.HBM)],
        out_specs=pl.BlockSpec(memory_space=pltpu.MemorySpace.HBM),
        scratch_shapes=[
            pltpu.VMEM((2, chunk, n), jnp.float32),
            pltpu.VMEM((2, chunk, n), jnp.bfloat16),
            pltpu.VMEM((2, chunk, n), jnp.bfloat16),
            pltpu.VMEM((2, chunk, n), jnp.bfloat16),
            pltpu.SemaphoreType.DMA((2,)),
            pltpu.SemaphoreType.DMA((2,)),
            pltpu.SemaphoreType.DMA((2,)),
            pltpu.SemaphoreType.DMA((2,)),
            pltpu.SemaphoreType.DMA((2,)),
            pltpu.SemaphoreType.DMA((2,)),
            pltpu.SemaphoreType.REGULAR,
            pltpu.SemaphoreType.REGULAR,
        ],
    )(x)


# device time: 223813 ns/iter; 1.9412x vs baseline; 1.0538x over previous
import jax
import jax.numpy as jnp
from jax import lax
from jax.experimental import pallas as pl
from jax.experimental.pallas import tpu as pltpu

N_CHUNKS = 16


def kernel(x):
    m, n = x.shape
    half = m // 2
    chunk = half // N_CHUNKS
    N = N_CHUNKS

    def body(x_hbm, out_hbm, in_vmem, send_x, recv_x, sum_buf,
             load_sems, store_sems, sx_send, sx_recv, sy_send, sy_recv,
             credit_x, credit_y):
        my_x = lax.axis_index("x")
        my_y = lax.axis_index("y")
        row0 = my_y * half
        x_peer = (1 - my_x, my_y)
        y_peer = (my_x, 1 - my_y)

        def load(c):
            return pltpu.make_async_copy(
                x_hbm.at[pl.ds(row0 + c * chunk, chunk), :],
                in_vmem.at[c % 2], load_sems.at[c % 2])

        def cast(c):
            send_x[c % 2] = in_vmem[c % 2].astype(jnp.bfloat16)

        def rdma_x(c):
            return pltpu.make_async_remote_copy(
                src_ref=send_x.at[c % 2], dst_ref=recv_x.at[c % 2],
                send_sem=sx_send.at[c % 2], recv_sem=sx_recv.at[c % 2],
                device_id=x_peer, device_id_type=pl.DeviceIdType.MESH)

        def rdma_y(c):
            return pltpu.make_async_remote_copy(
                src_ref=sum_buf.at[c % 2],
                dst_ref=out_hbm.at[pl.ds(row0 + c * chunk, chunk), :],
                send_sem=sy_send.at[c % 2], recv_sem=sy_recv.at[c % 2],
                device_id=y_peer, device_id_type=pl.DeviceIdType.MESH)

        def store(c):
            return pltpu.make_async_copy(
                sum_buf.at[c % 2],
                out_hbm.at[pl.ds(row0 + c * chunk, chunk), :],
                store_sems.at[c % 2])

        load(0).start()
        load(0).wait()
        cast(0)
        load(1).start()
        rdma_x(0).start()
        load(1).wait()
        cast(1)
        load(2).start()

        for c in range(N):
            if c + 1 < N:
                if c + 1 >= 2:
                    pl.semaphore_wait(credit_x, 1)
                rdma_x(c + 1).start()

            rdma_x(c).wait_recv()

            if c >= 2:
                rdma_y(c - 2).wait_send()
                store(c - 2).wait()

            sum_buf[c % 2] = send_x[c % 2] + recv_x[c % 2]
            if c <= N - 3:
                pl.semaphore_signal(
                    credit_x, inc=1, device_id=x_peer,
                    device_id_type=pl.DeviceIdType.MESH)

            if c >= 2:
                pl.semaphore_wait(credit_y, 1)
            store(c).start()
            rdma_y(c).start()

            if c >= 1:
                rdma_y(c - 1).wait_recv()
                if c - 1 <= N - 3:
                    pl.semaphore_signal(
                        credit_y, inc=1, device_id=y_peer,
                        device_id_type=pl.DeviceIdType.MESH)

            if c + 2 < N:
                load(c + 2).wait()
                rdma_x(c).wait_send()
                cast(c + 2)
                if c + 3 < N:
                    load(c + 3).start()

        rdma_y(N - 1).wait_recv()
        rdma_x(N - 2).wait_send()
        rdma_x(N - 1).wait_send()
        rdma_y(N - 2).wait_send()
        rdma_y(N - 1).wait_send()
        store(N - 2).wait()
        store(N - 1).wait()

    return pl.pallas_call(
        body,
        out_shape=jax.ShapeDtypeStruct((m, n), jnp.bfloat16),
        in_specs=[pl.BlockSpec(memory_space=pltpu.MemorySpace.HBM)],
        out_specs=pl.BlockSpec(memory_space=pltpu.MemorySpace.HBM),
        scratch_shapes=[
            pltpu.VMEM((2, chunk, n), jnp.float32),
            pltpu.VMEM((2, chunk, n), jnp.bfloat16),
            pltpu.VMEM((2, chunk, n), jnp.bfloat16),
            pltpu.VMEM((2, chunk, n), jnp.bfloat16),
            pltpu.SemaphoreType.DMA((2,)),
            pltpu.SemaphoreType.DMA((2,)),
            pltpu.SemaphoreType.DMA((2,)),
            pltpu.SemaphoreType.DMA((2,)),
            pltpu.SemaphoreType.DMA((2,)),
            pltpu.SemaphoreType.DMA((2,)),
            pltpu.SemaphoreType.REGULAR,
            pltpu.SemaphoreType.REGULAR,
        ],
    )(x)
